# baseline (speedup 1.0000x reference)
import numpy as np

import jax
import jax.numpy as jnp
from jax import lax
from jax.experimental import pallas as pl
from jax.experimental.pallas import tpu as pltpu
from jax.experimental.pallas import tpu_sc as plsc

T, LAT, LON, NQ = 64, 512, 1024, 100000

NC, NS, L = 2, 16, 16
NW = NC * NS
PER_W = 3200
G = 128
NG = PER_W // G
NV = PER_W // L
NQP = NW * PER_W


def _nearest_on_axis(n, q):
    step = jnp.float32(np.float32(1.0) / np.float32(n - 1))

    def axv(j):
        return j.astype(jnp.float32) * step

    x = q * jnp.float32(n - 1)
    cand = jnp.clip(x.astype(jnp.int32) + 1, 1, n - 1)
    cand = jnp.where((cand > 1) & (axv(cand - 1) >= q), cand - 1, cand)
    cand = jnp.where((cand < n - 1) & (axv(cand) < q), cand + 1, cand)
    left = axv(cand - 1)
    right = axv(cand)
    return jnp.where((q - left) <= (right - q), cand - 1, cand)


def _sc_body(tq_h, laq_h, loq_h, u_h, v_h, m_h,
             out_h,
             tq, laq, loq,
             idx_uv, idx_m, gu, gv, gm, sem):
    wid = lax.axis_index("s") * NC + lax.axis_index("c")
    base = wid * PER_W

    pltpu.sync_copy(tq_h.at[pl.ds(base, PER_W)], tq)
    pltpu.sync_copy(laq_h.at[pl.ds(base, PER_W)], laq)
    pltpu.sync_copy(loq_h.at[pl.ds(base, PER_W)], loq)

    def chunk(j, _):
        for k in range(G // L):
            s = pl.ds(j * G + k * L, L)
            t_i = _nearest_on_axis(T, tq[s])
            la_i = _nearest_on_axis(LAT, laq[s])
            lo_i = _nearest_on_axis(LON, loq[s])
            idx_m[s] = (la_i << 10) | lo_i
            idx_uv[s] = ((t_i << 19) | ((la_i >> 3) << 13)
                         | ((lo_i >> 7) << 10) | ((la_i & 7) << 7)
                         | (lo_i & 127))
        s = pl.ds(j * G, G)
        pltpu.async_copy(u_h.at[idx_uv.at[s]], gu.at[s], sem)
        pltpu.async_copy(v_h.at[idx_uv.at[s]], gv.at[s], sem)
        pltpu.async_copy(m_h.at[idx_m.at[s]], gm.at[s], sem)
        return 0

    lax.fori_loop(0, NG, chunk, 0, unroll=False)

    pltpu.make_async_copy(u_h.at[pl.ds(0, PER_W)], gu, sem).wait()
    pltpu.make_async_copy(u_h.at[pl.ds(0, PER_W)], gv, sem).wait()
    pltpu.make_async_copy(u_h.at[pl.ds(0, PER_W)], gm, sem).wait()

    def apply_mask(i, _):
        s = pl.ds(i * L, L)
        hit = gm[s] != 0
        gu[s] = jnp.where(hit, jnp.float32(0.0), gu[s])
        gv[s] = jnp.where(hit, jnp.float32(0.0), gv[s])
        return 0

    lax.fori_loop(0, NV, apply_mask, 0, unroll=False)

    pltpu.sync_copy(gu, out_h.at[pl.ds(base, PER_W)])
    pltpu.sync_copy(gv, out_h.at[pl.ds(NQP + base, PER_W)])


@jax.jit
def _run(tq, laq, loq, u, v, m_flat):
    mesh = plsc.VectorSubcoreMesh(core_axis_name="c", subcore_axis_name="s")
    f = pl.kernel(
        _sc_body,
        out_type=jax.ShapeDtypeStruct((2 * NQP,), jnp.float32),
        mesh=mesh,
        scratch_types=[
            pltpu.VMEM((PER_W,), jnp.float32),
            pltpu.VMEM((PER_W,), jnp.float32),
            pltpu.VMEM((PER_W,), jnp.float32),
            pltpu.VMEM((PER_W,), jnp.int32),
            pltpu.VMEM((PER_W,), jnp.int32),
            pltpu.VMEM((PER_W,), jnp.float32),
            pltpu.VMEM((PER_W,), jnp.float32),
            pltpu.VMEM((PER_W,), jnp.int32),
            pltpu.SemaphoreType.DMA,
        ],
    )
    return f(tq, laq, loq, u, v, m_flat)


def kernel(time_q, latitude_q, longitude_q, time_axis, lat_axis, lon_axis,
           u, v, mask):
    def tile_order_flat(x):
        return (x.reshape(T, LAT // 8, 8, LON // 128, 128)
                .transpose(0, 1, 3, 2, 4).reshape(-1))

    pad = NQP - NQ

    def padq(q):
        return jnp.pad(q, (0, pad))

    out = _run(padq(time_q), padq(latitude_q), padq(longitude_q),
               tile_order_flat(u), tile_order_flat(v),
               mask.astype(jnp.int32).reshape(-1))
    return jnp.stack([out[:NQ], out[NQP:NQP + NQ]])


# speedup vs baseline: 1.3260x; 1.3260x over previous
import numpy as np

import jax
import jax.numpy as jnp
from jax import lax
from jax.experimental import pallas as pl
from jax.experimental.pallas import tpu as pltpu
from jax.experimental.pallas import tpu_sc as plsc

T, LAT, LON, NQ = 64, 512, 1024, 100000

NC, NS, L = 2, 16, 16
NW = NC * NS
PER_W = 3200
G = 128
NG = PER_W // G
NV = PER_W // L
TAIL = NQ - (NW - 1) * PER_W
FILL = PER_W - TAIL


def _nearest_on_axis(n, q):
    step = jnp.float32(np.float32(1.0) / np.float32(n - 1))

    def axv(j):
        return j.astype(jnp.float32) * step

    x = q * jnp.float32(n - 1)
    cand = jnp.clip(x.astype(jnp.int32) + 1, 1, n - 1)
    cand = jnp.where((cand > 1) & (axv(cand - 1) >= q), cand - 1, cand)
    cand = jnp.where((cand < n - 1) & (axv(cand) < q), cand + 1, cand)
    left = axv(cand - 1)
    right = axv(cand)
    return jnp.where((q - left) <= (right - q), cand - 1, cand)


def _sc_body(tq_h, laq_h, loq_h, u_h, v_h, m_h,
             out_h,
             tq, laq, loq,
             idx_uv, idx_m, gu, gv, gm, sem):
    wid = lax.axis_index("s") * NC + lax.axis_index("c")
    base = wid * PER_W
    is_tail = wid == NW - 1

    @pl.when(jnp.logical_not(is_tail))
    def _():
        pltpu.sync_copy(tq_h.at[pl.ds(base, PER_W)], tq)
        pltpu.sync_copy(laq_h.at[pl.ds(base, PER_W)], laq)
        pltpu.sync_copy(loq_h.at[pl.ds(base, PER_W)], loq)

    @pl.when(is_tail)
    def _():
        tb = (NW - 1) * PER_W
        pltpu.sync_copy(tq_h.at[pl.ds(tb, TAIL)], tq.at[pl.ds(0, TAIL)])
        pltpu.sync_copy(laq_h.at[pl.ds(tb, TAIL)], laq.at[pl.ds(0, TAIL)])
        pltpu.sync_copy(loq_h.at[pl.ds(tb, TAIL)], loq.at[pl.ds(0, TAIL)])
        pltpu.sync_copy(tq_h.at[pl.ds(0, FILL)], tq.at[pl.ds(TAIL, FILL)])
        pltpu.sync_copy(laq_h.at[pl.ds(0, FILL)], laq.at[pl.ds(TAIL, FILL)])
        pltpu.sync_copy(loq_h.at[pl.ds(0, FILL)], loq.at[pl.ds(TAIL, FILL)])

    def chunk(j, _):
        for k in range(G // L):
            s = pl.ds(j * G + k * L, L)
            t_i = _nearest_on_axis(T, tq[s])
            la_i = _nearest_on_axis(LAT, laq[s])
            lo_i = _nearest_on_axis(LON, loq[s])
            idx_m[s] = (la_i << 10) | lo_i
            idx_uv[s] = ((t_i << 19) | ((la_i >> 3) << 13)
                         | ((lo_i >> 7) << 10) | ((la_i & 7) << 7)
                         | (lo_i & 127))
        s = pl.ds(j * G, G)
        pltpu.async_copy(u_h.at[idx_uv.at[s]], gu.at[s], sem)
        pltpu.async_copy(v_h.at[idx_uv.at[s]], gv.at[s], sem)
        pltpu.async_copy(m_h.at[idx_m.at[s]], gm.at[s], sem)
        return 0

    lax.fori_loop(0, NG, chunk, 0, unroll=False)

    pltpu.make_async_copy(u_h.at[pl.ds(0, PER_W)], gu, sem).wait()
    pltpu.make_async_copy(u_h.at[pl.ds(0, PER_W)], gv, sem).wait()
    pltpu.make_async_copy(u_h.at[pl.ds(0, PER_W)], gm, sem).wait()

    def apply_mask(i, _):
        s = pl.ds(i * L, L)
        hit = gm[s] != 0
        gu[s] = jnp.where(hit, jnp.float32(0.0), gu[s])
        gv[s] = jnp.where(hit, jnp.float32(0.0), gv[s])
        return 0

    lax.fori_loop(0, NV, apply_mask, 0, unroll=False)

    @pl.when(jnp.logical_not(is_tail))
    def _():
        pltpu.sync_copy(gu, out_h.at[pl.ds(base, PER_W)])
        pltpu.sync_copy(gv, out_h.at[pl.ds(NQ + base, PER_W)])

    @pl.when(is_tail)
    def _():
        tb = (NW - 1) * PER_W
        pltpu.sync_copy(gu.at[pl.ds(0, TAIL)], out_h.at[pl.ds(tb, TAIL)])
        pltpu.sync_copy(gv.at[pl.ds(0, TAIL)], out_h.at[pl.ds(NQ + tb, TAIL)])


@jax.jit
def _run(tq, laq, loq, u, v, m_flat):
    mesh = plsc.VectorSubcoreMesh(core_axis_name="c", subcore_axis_name="s")
    f = pl.kernel(
        _sc_body,
        out_type=jax.ShapeDtypeStruct((2 * NQ,), jnp.float32),
        mesh=mesh,
        scratch_types=[
            pltpu.VMEM((PER_W,), jnp.float32),
            pltpu.VMEM((PER_W,), jnp.float32),
            pltpu.VMEM((PER_W,), jnp.float32),
            pltpu.VMEM((PER_W,), jnp.int32),
            pltpu.VMEM((PER_W,), jnp.int32),
            pltpu.VMEM((PER_W,), jnp.float32),
            pltpu.VMEM((PER_W,), jnp.float32),
            pltpu.VMEM((PER_W,), jnp.int32),
            pltpu.SemaphoreType.DMA,
        ],
    )
    return f(tq, laq, loq, u, v, m_flat)


def kernel(time_q, latitude_q, longitude_q, time_axis, lat_axis, lon_axis,
           u, v, mask):
    def tile_order_flat(x):
        return (x.reshape(T, LAT // 8, 8, LON // 128, 128)
                .transpose(0, 1, 3, 2, 4).reshape(-1))

    out = _run(time_q, latitude_q, longitude_q,
               tile_order_flat(u), tile_order_flat(v),
               mask.astype(jnp.int32).reshape(-1))
    return out.reshape(2, NQ)
